# Initial kernel scaffold; baseline (speedup 1.0000x reference)
import functools

import jax
import jax.numpy as jnp
from jax import lax
from jax.experimental import pallas as pl
from jax.experimental.pallas import tpu as pltpu
from jax.experimental.pallas import tpu_sc as plsc

NUM_EMB = 1000000
DIM = 32
B_TOTAL = 16384 * 26
NUM_WORKERS = 32
BPW = B_TOTAL // NUM_WORKERS
CHUNK = 1024
NCHUNK = BPW // CHUNK

_mesh = plsc.VectorSubcoreMesh(core_axis_name="c", subcore_axis_name="s")


@functools.partial(
    pl.kernel,
    mesh=_mesh,
    out_type=jax.ShapeDtypeStruct((B_TOTAL, DIM), jnp.float32),
    scratch_types=[
        pltpu.VMEM((BPW,), jnp.int32),
        pltpu.VMEM((CHUNK, DIM), jnp.float32),
        pltpu.SemaphoreType.DMA,
    ],
)
def _gather_kernel(idx_hbm, table_hbm, out_hbm, idx_v, rows_v, sem):
    wid = lax.axis_index("s") * 2 + lax.axis_index("c")
    base = wid * BPW
    pltpu.sync_copy(idx_hbm.at[pl.ds(base, BPW)], idx_v)

    def body(g, carry):
        off = g * CHUNK
        pltpu.async_copy(
            table_hbm.at[idx_v.at[pl.ds(off, CHUNK)]], rows_v, sem
        ).wait()
        pltpu.sync_copy(rows_v, out_hbm.at[pl.ds(base + off, CHUNK)])
        return carry

    lax.fori_loop(0, NCHUNK, body, 0)


def kernel(token_ids, weight):
    flat = token_ids.reshape(-1).astype(jnp.int32)
    out = _gather_kernel(flat, weight)
    return out.reshape(token_ids.shape + (DIM,))


# speedup vs baseline: 1.5597x; 1.5597x over previous
import functools

import jax
import jax.numpy as jnp
from jax import lax
from jax.experimental import pallas as pl
from jax.experimental.pallas import tpu as pltpu
from jax.experimental.pallas import tpu_sc as plsc

NUM_EMB = 1000000
DIM = 32
B_TOTAL = 16384 * 26
NUM_WORKERS = 32
BPW = B_TOTAL // NUM_WORKERS
CHUNK = 1024
NCHUNK = BPW // CHUNK

_mesh = plsc.VectorSubcoreMesh(core_axis_name="c", subcore_axis_name="s")


@functools.partial(
    pl.kernel,
    mesh=_mesh,
    compiler_params=pltpu.CompilerParams(use_tc_tiling_on_sc=False),
    out_type=jax.ShapeDtypeStruct((B_TOTAL, DIM), jnp.float32),
    scratch_types=[
        pltpu.VMEM((BPW,), jnp.int32),
        pltpu.VMEM((CHUNK, DIM), jnp.float32),
        pltpu.SemaphoreType.DMA,
    ],
)
def _gather_kernel(idx_hbm, table_hbm, out_hbm, idx_v, rows_v, sem):
    wid = lax.axis_index("s") * 2 + lax.axis_index("c")
    base = wid * BPW
    pltpu.sync_copy(idx_hbm.at[pl.ds(base, BPW)], idx_v)

    def body(g, carry):
        off = g * CHUNK
        pltpu.async_copy(
            table_hbm.at[idx_v.at[pl.ds(off, CHUNK)]], rows_v, sem
        ).wait()
        pltpu.sync_copy(rows_v, out_hbm.at[pl.ds(base + off, CHUNK)])
        return carry

    lax.fori_loop(0, NCHUNK, body, 0)


def kernel(token_ids, weight):
    flat = token_ids.reshape(-1).astype(jnp.int32)
    out = _gather_kernel(flat, weight)
    return out.reshape(token_ids.shape + (DIM,))


# speedup vs baseline: 1.5738x; 1.0090x over previous
import functools

import jax
import jax.numpy as jnp
from jax import lax
from jax.experimental import pallas as pl
from jax.experimental.pallas import tpu as pltpu
from jax.experimental.pallas import tpu_sc as plsc

NUM_EMB = 1000000
DIM = 32
B_TOTAL = 16384 * 26
NUM_WORKERS = 32
BPW = B_TOTAL // NUM_WORKERS
CHUNK = 512
NCHUNK = BPW // CHUNK
NBUF = 4

_mesh = plsc.VectorSubcoreMesh(core_axis_name="c", subcore_axis_name="s")


@functools.partial(
    pl.kernel,
    mesh=_mesh,
    compiler_params=pltpu.CompilerParams(use_tc_tiling_on_sc=False),
    out_type=jax.ShapeDtypeStruct((B_TOTAL, DIM), jnp.float32),
    scratch_types=[
        pltpu.VMEM((BPW,), jnp.int32),
        [pltpu.VMEM((CHUNK, DIM), jnp.float32) for _ in range(NBUF)],
        [pltpu.SemaphoreType.DMA for _ in range(NBUF)],
        [pltpu.SemaphoreType.DMA for _ in range(NBUF)],
    ],
)
def _gather_kernel(idx_hbm, table_hbm, out_hbm, idx_v, bufs, gsems, osems):
    wid = lax.axis_index("s") * 2 + lax.axis_index("c")
    base = wid * BPW
    pltpu.sync_copy(idx_hbm.at[pl.ds(base, BPW)], idx_v)

    def start_gather(g):
        b = g % NBUF
        return pltpu.async_copy(
            table_hbm.at[idx_v.at[pl.ds(g * CHUNK, CHUNK)]], bufs[b], gsems[b]
        )

    def start_ocopy(g):
        b = g % NBUF
        return pltpu.async_copy(
            bufs[b], out_hbm.at[pl.ds(base + g * CHUNK, CHUNK)], osems[b]
        )

    gh = [start_gather(g) for g in range(NBUF)]
    oh = [None] * NBUF
    for g in range(NCHUNK):
        b = g % NBUF
        gh[b].wait()
        oh[b] = start_ocopy(g)
        nxt = g + NBUF
        if nxt < NCHUNK:
            oh[b].wait()
            gh[b] = start_gather(nxt)
    for g in range(NCHUNK - NBUF, NCHUNK):
        oh[g % NBUF].wait()


def kernel(token_ids, weight):
    flat = token_ids.reshape(-1).astype(jnp.int32)
    out = _gather_kernel(flat, weight)
    return out.reshape(token_ids.shape + (DIM,))
